# baseline (speedup 1.0000x reference)
import functools
import jax
import jax.numpy as jnp
from jax import lax
from jax.experimental import pallas as pl
from jax.experimental.pallas import tpu as pltpu

X_LEN = 50
D = 32
F = 26
FL = F * X_LEN
B = 1024
BBm = 8
NBUF = 4
STEPS = B // BBm


def _ln(x, t, g, bta):
    w = x + t[None, :, :]
    mean = jnp.mean(w, axis=1, keepdims=True)
    var = jnp.mean(w * w, axis=1, keepdims=True) - mean * mean
    rs = lax.rsqrt(var + 1e-5)
    return (w - mean) * (rs * g) + bta


def _body(x_hbm, t_ref, g_ref, b_ref, o_hbm, xv, ov, si, so):
    t = t_ref[...]
    g = g_ref[...].reshape(1, D, 1)
    bta = b_ref[...].reshape(1, D, 1)

    def in_copy(slot, step):
        return pltpu.make_async_copy(
            x_hbm.at[pl.ds(step * BBm, BBm)], xv.at[slot], si.at[slot])

    def out_copy(slot, step):
        return pltpu.make_async_copy(
            ov.at[slot], o_hbm.at[pl.ds(step * BBm, BBm)], so.at[slot])

    for i in range(NBUF):
        in_copy(i, i).start()

    def step_body(tstep, carry):
        slot = lax.rem(tstep, NBUF)
        in_copy(slot, 0).wait()
        x = xv[slot]
        y = _ln(x, t, g, bta)

        @pl.when(tstep >= NBUF)
        def _():
            out_copy(slot, 0).wait()

        ov[slot] = jnp.transpose(y, (0, 2, 1))
        out_copy(slot, tstep).start()

        @pl.when(tstep + NBUF < STEPS)
        def _():
            in_copy(slot, tstep + NBUF).start()

        return carry

    lax.fori_loop(0, STEPS, step_body, 0)
    for i in range(NBUF):
        out_copy(i, 0).wait()


def kernel(x, table, gamma, beta, batch_size):
    batch = x.shape[0]
    resid = (jnp.asarray(batch_size, jnp.int32) - batch).astype(jnp.float32)
    beta_eff = beta + resid
    x3 = x.reshape(batch, D, FL)
    tfl = jnp.tile(table.T[:, None, :], (1, F, 1)).reshape(D, FL)
    out = pl.pallas_call(
        _body,
        in_specs=[
            pl.BlockSpec(memory_space=pl.ANY),
            pl.BlockSpec(memory_space=pltpu.VMEM),
            pl.BlockSpec(memory_space=pltpu.VMEM),
            pl.BlockSpec(memory_space=pltpu.VMEM),
        ],
        out_specs=pl.BlockSpec(memory_space=pl.ANY),
        out_shape=jax.ShapeDtypeStruct((batch, FL, D), jnp.float32),
        scratch_shapes=[
            pltpu.VMEM((NBUF, BBm, D, FL), jnp.float32),
            pltpu.VMEM((NBUF, BBm, FL, D), jnp.float32),
            pltpu.SemaphoreType.DMA((NBUF,)),
            pltpu.SemaphoreType.DMA((NBUF,)),
        ],
        compiler_params=pltpu.CompilerParams(
            vmem_limit_bytes=100 * 1024 * 1024,
        ),
    )(x3, tfl, gamma, beta_eff)
    return out.reshape(batch, F, X_LEN, D)
